# baseline (speedup 1.0000x reference)
import functools

import jax
import jax.numpy as jnp
from jax import lax
from jax.experimental import pallas as pl
from jax.experimental.pallas import tpu as pltpu
from jax.experimental.pallas import tpu_sc as plsc

N = 2048
IN = 128
D = 64
H = 4
E = 65536
H1 = 256
H2 = 64
SLOPE = 0.2
HD = H * D

NB = 256
FB = 8192
CB = 256

NC = 2
NS = 16
L = 16
NW = NC * NS
EPW = E // NW
K = 80
NCHUNK = EPW // K
RPT = N // NS


def _rne_bf16_tc(v):
    u = lax.bitcast_convert_type(v, jnp.uint32)
    r = (u + jnp.uint32(0x7FFF) + ((u >> jnp.uint32(16)) & jnp.uint32(1))
         ) & jnp.uint32(0xFFFF0000)
    return lax.bitcast_convert_type(r, jnp.float32)


def _embed_body(pe_ref, win_ref, bin_ref, wl_ref, bl_ref, wr_ref, br_ref,
                xl_ref, xr_ref):
    prod = (pe_ref[...].astype(jnp.float32)[:, :, None]
            * _rne_bf16_tc(win_ref[...]))
    x = jnp.sum(prod, axis=1) + bin_ref[...]
    x = jnp.maximum(x, 0.0).astype(jnp.bfloat16)
    xl_ref[...] = jnp.dot(x, wl_ref[...],
                          preferred_element_type=jnp.float32) + bl_ref[...]
    xr_ref[...] = jnp.dot(x, wr_ref[...],
                          preferred_element_type=jnp.float32) + br_ref[...]


def _embed(pe_bf, win_bf, b_in, wl_bf, bl, wr_bf, br):
    return pl.pallas_call(
        _embed_body,
        grid=(N // NB,),
        in_specs=[
            pl.BlockSpec((NB, IN), lambda i: (i, 0)),
            pl.BlockSpec((NB, IN, D), lambda i: (i, 0, 0)),
            pl.BlockSpec((NB, D), lambda i: (i, 0)),
            pl.BlockSpec((D, HD), lambda i: (0, 0)),
            pl.BlockSpec((1, HD), lambda i: (0, 0)),
            pl.BlockSpec((D, HD), lambda i: (0, 0)),
            pl.BlockSpec((1, HD), lambda i: (0, 0)),
        ],
        out_specs=[
            pl.BlockSpec((NB, HD), lambda i: (i, 0)),
            pl.BlockSpec((NB, HD), lambda i: (i, 0)),
        ],
        out_shape=[
            jax.ShapeDtypeStruct((N, HD), jnp.float32),
            jax.ShapeDtypeStruct((N, HD), jnp.float32),
        ],
    )(pe_bf, win_bf, b_in, wl_bf, bl.reshape(1, -1), wr_bf, br.reshape(1, -1))


def _rne_bf16(v):
    u = lax.bitcast_convert_type(v, jnp.uint32)
    r = (u + jnp.uint32(0x7FFF) + ((u >> jnp.uint32(16)) & jnp.uint32(1))
         ) & jnp.uint32(0xFFFF0000)
    return lax.bitcast_convert_type(r, jnp.float32)


NPT = N // NW
LMAX = 2800
FCH = 4096


def _edge_sc_body(src_hbm, dst_hbm, xl_hbm, xr_hbm, att_hbm,
                  num_out, den_out,
                  sbuf, dbuf, pkl, srcc0, dstc0, dlc0, srcc1, dstc1, dlc1,
                  xl0, xr0, xl1, xr1,
                  num_t, den_t, att_v, sl0, sr0, sl1, sr1):
    cid = lax.axis_index("c")
    sid = lax.axis_index("s")
    wid = sid * NC + cid

    pltpu.sync_copy(att_hbm, att_v)
    att = [att_v[pl.ds(L * j, L)] for j in range(HD // L)]
    lane = lax.iota(jnp.int32, L)
    perms = [lane ^ (1 << p) for p in range(4)]
    gdn = lax.GatherDimensionNumbers(offset_dims=(), collapsed_slice_dims=(0,),
                                     start_index_map=(0,))

    def _allsum(v):
        for p in perms:
            v = v + lax.gather(v, p[:, None], gdn, (1,),
                               mode=lax.GatherScatterMode.PROMISE_IN_BOUNDS)
        return v

    zf = jnp.zeros((L,), jnp.float32)

    def zrow(r, _):
        for j in range(HD // L):
            num_t[r, pl.ds(L * j, L)] = zf
        den_t[r, :] = zf
        return 0

    lax.fori_loop(0, NPT + 1, zrow, 0)

    def fchunk(fc, cnt):
        pltpu.sync_copy(src_hbm.at[pl.ds(fc * FCH, FCH)], sbuf)
        pltpu.sync_copy(dst_hbm.at[pl.ds(fc * FCH, FCH)], dbuf)

        def fvec(i, cnt):
            s = sbuf[pl.ds(i * L, L)]
            dg = dbuf[pl.ds(i * L, L)]
            msk = (dg >> 6) == wid
            key = jnp.where(msk, lane, lane + L)
            _, vs = plsc.sort_key_val(key, s | (dg << 11))
            pkl[pl.ds(cnt, L)] = vs
            return cnt + plsc.all_reduce_population_count(msk)[0]

        return lax.fori_loop(0, FCH // L, fvec, cnt)

    cnt = lax.fori_loop(0, E // FCH, fchunk, jnp.int32(0))

    pad_v = jnp.full((L,), 4095 << 11, jnp.int32)
    for t in range(3 * K // L):
        pkl[pl.ds(cnt + L * t, L)] = pad_v

    wbase = wid * NPT

    def unpack(ci, srcc, dstc, dlc):
        co = ci * K
        for j in range(K // L):
            v = pkl[pl.ds(co + L * j, L)]
            dg = v >> 11
            srcc[pl.ds(L * j, L)] = v & 2047
            dstc[pl.ds(L * j, L)] = jnp.minimum(dg, N - 1)
            dlc[pl.ds(L * j, L)] = jnp.minimum(dg - wbase, NPT)

    def issue(srcc, dstc, xlb, xrb, sl, sr):
        pltpu.async_copy(xl_hbm.at[srcc], xlb, sl)
        pltpu.async_copy(xr_hbm.at[dstc], xrb, sr)

    def wait(srcc, dstc, xlb, xrb, sl, sr):
        pltpu.make_async_copy(xl_hbm.at[srcc], xlb, sl).wait()
        pltpu.make_async_copy(xr_hbm.at[dstc], xrb, sr).wait()

    def compute(dlc, xlr, xrr):
        def group_body(g, _):
            g4 = g * 4
            dlv = dlc[pl.ds(g4, L)]
            for e in range(4):
                k = g4 + e
                dl = dlv[e]
                a = [xlr[k, pl.ds(L * j, L)] for j in range(HD // L)]
                b = [xrr[k, pl.ds(L * j, L)] for j in range(HD // L)]
                w = []
                for h in range(H):
                    acc = jnp.zeros((L,), jnp.float32)
                    for j in range(H * h, H * (h + 1)):
                        m = a[j] + b[j]
                        lm = jnp.maximum(m, SLOPE * m)
                        acc = acc + _rne_bf16(lm) * att[j]
                    w.append(jnp.exp(_allsum(acc)))
                for h in range(H):
                    for j in range(H * h, H * (h + 1)):
                        plsc.addupdate(num_t.at[dl, pl.ds(L * j, L)],
                                       a[j] * w[h])
                dv = jnp.where(lane == 0, w[0], 0.0)
                dv = jnp.where(lane == 1, w[1], dv)
                dv = jnp.where(lane == 2, w[2], dv)
                dv = jnp.where(lane == 3, w[3], dv)
                plsc.addupdate(den_t.at[dl, :], dv)
            return 0

        lax.fori_loop(0, K // 4, group_body, 0)

    nch = (cnt + K - 1) // K
    npair = (nch + 1) // 2
    unpack(0, srcc0, dstc0, dlc0)
    issue(srcc0, dstc0, xl0, xr0, sl0, sr0)

    def pair_body(g, _):
        c0 = 2 * g
        unpack(c0 + 1, srcc1, dstc1, dlc1)
        issue(srcc1, dstc1, xl1, xr1, sl1, sr1)
        wait(srcc0, dstc0, xl0, xr0, sl0, sr0)
        compute(dlc0, xl0, xr0)
        unpack(c0 + 2, srcc0, dstc0, dlc0)
        issue(srcc0, dstc0, xl0, xr0, sl0, sr0)
        wait(srcc1, dstc1, xl1, xr1, sl1, sr1)
        compute(dlc1, xl1, xr1)
        return 0

    lax.fori_loop(0, npair, pair_body, 0)
    wait(srcc0, dstc0, xl0, xr0, sl0, sr0)

    pltpu.sync_copy(num_t.at[pl.ds(0, NPT)], num_out.at[pl.ds(wid * NPT, NPT)])
    pltpu.sync_copy(den_t.at[pl.ds(0, NPT)], den_out.at[pl.ds(wid * NPT, NPT)])


def _edge_sc(src, dst, xl, xr, att_r):
    mesh = plsc.VectorSubcoreMesh(core_axis_name="c", subcore_axis_name="s")
    k = pl.kernel(
        _edge_sc_body,
        out_type=[
            jax.ShapeDtypeStruct((N, HD), jnp.float32),
            jax.ShapeDtypeStruct((N, L), jnp.float32),
        ],
        mesh=mesh,
        compiler_params=pltpu.CompilerParams(needs_layout_passes=False),
        scratch_types=[
            pltpu.VMEM((FCH,), jnp.int32),
            pltpu.VMEM((FCH,), jnp.int32),
            pltpu.VMEM((LMAX,), jnp.int32),
            pltpu.VMEM((K,), jnp.int32),
            pltpu.VMEM((K,), jnp.int32),
            pltpu.VMEM((K + L,), jnp.int32),
            pltpu.VMEM((K,), jnp.int32),
            pltpu.VMEM((K,), jnp.int32),
            pltpu.VMEM((K + L,), jnp.int32),
            pltpu.VMEM((K, HD), jnp.float32),
            pltpu.VMEM((K, HD), jnp.float32),
            pltpu.VMEM((K, HD), jnp.float32),
            pltpu.VMEM((K, HD), jnp.float32),
            pltpu.VMEM((NPT + 1, HD), jnp.float32),
            pltpu.VMEM((NPT + 1, L), jnp.float32),
            pltpu.VMEM((HD,), jnp.float32),
            pltpu.SemaphoreType.DMA,
            pltpu.SemaphoreType.DMA,
            pltpu.SemaphoreType.DMA,
            pltpu.SemaphoreType.DMA,
        ],
    )
    return k(src, dst, xl, xr, att_r)


def _combine_body(num_ref, den_ref, gb_ref, out_ref):
    num = num_ref[...]
    den = den_ref[...]
    acc = jnp.zeros((CB, D), jnp.float32)
    for h in range(H):
        nh = num[:, h * D:(h + 1) * D]
        dh = den[:, h:h + 1] + 1e-16
        acc = acc + nh / dh
    out = acc * (1.0 / H) + gb_ref[...]
    out_ref[...] = jnp.maximum(out, SLOPE * out)


def _combine(num, den, gat_bias):
    return pl.pallas_call(
        _combine_body,
        grid=(N // CB,),
        in_specs=[
            pl.BlockSpec((CB, HD), lambda i: (i, 0)),
            pl.BlockSpec((CB, L), lambda i: (i, 0)),
            pl.BlockSpec((1, D), lambda i: (0, 0)),
        ],
        out_specs=pl.BlockSpec((CB, D), lambda i: (i, 0)),
        out_shape=jax.ShapeDtypeStruct((N, D), jnp.float32),
    )(num, den, gat_bias.reshape(1, D))


def _mlp_body(flat_ref, w1_ref, b1_ref, w2_ref, b2_ref, w3_ref, b3_ref,
              y_ref, acc_ref):
    i = pl.program_id(0)
    fb = flat_ref[...].astype(jnp.bfloat16)
    part = jnp.dot(fb, w1_ref[...].astype(jnp.bfloat16),
                   preferred_element_type=jnp.float32)

    @pl.when(i == 0)
    def _():
        acc_ref[...] = jnp.zeros_like(acc_ref)

    acc_ref[...] += part

    @pl.when(i == pl.num_programs(0) - 1)
    def _():
        h = jnp.maximum(acc_ref[...] + b1_ref[...], 0.0).astype(jnp.bfloat16)
        h2 = jnp.dot(h, w2_ref[...].astype(jnp.bfloat16),
                     preferred_element_type=jnp.float32) + b2_ref[...]
        h2 = jnp.maximum(h2, 0.0).astype(jnp.bfloat16)
        y_ref[...] = jnp.dot(h2, w3_ref[...].astype(jnp.bfloat16),
                             preferred_element_type=jnp.float32) + b3_ref[...]


def _mlp(flat, w1_bf, b1, w2_bf, b2, w3_bf, b3):
    y = pl.pallas_call(
        _mlp_body,
        grid=(N * D // FB,),
        in_specs=[
            pl.BlockSpec((1, FB), lambda i: (0, i)),
            pl.BlockSpec((FB, H1), lambda i: (i, 0)),
            pl.BlockSpec((1, H1), lambda i: (0, 0)),
            pl.BlockSpec((H1, H2), lambda i: (0, 0)),
            pl.BlockSpec((1, H2), lambda i: (0, 0)),
            pl.BlockSpec((H2, 1), lambda i: (0, 0)),
            pl.BlockSpec((1, 1), lambda i: (0, 0)),
        ],
        out_specs=pl.BlockSpec((1, 1), lambda i: (0, 0)),
        out_shape=jax.ShapeDtypeStruct((1, 1), jnp.float32),
        scratch_shapes=[pltpu.VMEM((1, H1), jnp.float32)],
    )(flat.reshape(1, N * D), w1_bf, b1.reshape(1, H1),
      w2_bf, b2.reshape(1, H2), w3_bf, b3.reshape(1, 1))
    return y.reshape(1)


def kernel(protein_embeddings, edge_index, W_in, b_in, Wl, bl, Wr, br, att,
           gat_bias, W1, b1, W2, b2, W3, b3):
    src = edge_index[0].astype(jnp.int32)
    dst = edge_index[1].astype(jnp.int32)
    bf = jnp.bfloat16
    xl, xr = _embed(protein_embeddings.astype(bf), W_in, b_in,
                    Wl.astype(bf), bl, Wr.astype(bf), br)
    att_r = att.astype(bf).astype(jnp.float32).reshape(HD)
    num, den = _edge_sc(src, dst, xl, xr, att_r)
    out = _combine(num, den, gat_bias)
    return _mlp(out, W1, b1, W2, b2, W3, b3)


# speedup vs baseline: 1.0955x; 1.0955x over previous
import functools

import jax
import jax.numpy as jnp
from jax import lax
from jax.experimental import pallas as pl
from jax.experimental.pallas import tpu as pltpu
from jax.experimental.pallas import tpu_sc as plsc

N = 2048
IN = 128
D = 64
H = 4
E = 65536
H1 = 256
H2 = 64
SLOPE = 0.2
HD = H * D

NB = 256
FB = 8192
CB = 256

NC = 2
NS = 16
L = 16
NW = NC * NS
EPW = E // NW
K = 128
NCHUNK = EPW // K
RPT = N // NS


def _rne_bf16_tc(v):
    u = lax.bitcast_convert_type(v, jnp.uint32)
    r = (u + jnp.uint32(0x7FFF) + ((u >> jnp.uint32(16)) & jnp.uint32(1))
         ) & jnp.uint32(0xFFFF0000)
    return lax.bitcast_convert_type(r, jnp.float32)


def _embed_body(pe_ref, win_ref, bin_ref, wl_ref, bl_ref, wr_ref, br_ref,
                xl_ref, xr_ref):
    prod = (pe_ref[...].astype(jnp.float32)[:, :, None]
            * _rne_bf16_tc(win_ref[...]))
    x = jnp.sum(prod, axis=1) + bin_ref[...]
    x = jnp.maximum(x, 0.0).astype(jnp.bfloat16)
    xl_ref[...] = jnp.dot(x, wl_ref[...],
                          preferred_element_type=jnp.float32) + bl_ref[...]
    xr_ref[...] = jnp.dot(x, wr_ref[...],
                          preferred_element_type=jnp.float32) + br_ref[...]


def _embed(pe_bf, win_bf, b_in, wl_bf, bl, wr_bf, br):
    return pl.pallas_call(
        _embed_body,
        grid=(N // NB,),
        in_specs=[
            pl.BlockSpec((NB, IN), lambda i: (i, 0)),
            pl.BlockSpec((NB, IN, D), lambda i: (i, 0, 0)),
            pl.BlockSpec((NB, D), lambda i: (i, 0)),
            pl.BlockSpec((D, HD), lambda i: (0, 0)),
            pl.BlockSpec((1, HD), lambda i: (0, 0)),
            pl.BlockSpec((D, HD), lambda i: (0, 0)),
            pl.BlockSpec((1, HD), lambda i: (0, 0)),
        ],
        out_specs=[
            pl.BlockSpec((NB, HD), lambda i: (i, 0)),
            pl.BlockSpec((NB, HD), lambda i: (i, 0)),
        ],
        out_shape=[
            jax.ShapeDtypeStruct((N, HD), jnp.float32),
            jax.ShapeDtypeStruct((N, HD), jnp.float32),
        ],
    )(pe_bf, win_bf, b_in, wl_bf, bl.reshape(1, -1), wr_bf, br.reshape(1, -1))


def _rne_bf16(v):
    u = lax.bitcast_convert_type(v, jnp.uint32)
    r = (u + jnp.uint32(0x8000)) & jnp.uint32(0xFFFF0000)
    return lax.bitcast_convert_type(r, jnp.float32)


NPT = N // NW
LMAX = 2704
FCH = 8192


def _edge_sc_body(src_hbm, dst_hbm, xl_hbm, xr_hbm, att_hbm,
                  num_out, den_out,
                  sbuf, dbuf, pkl, srcc, dstc, dlc, xl_rows, xr_rows,
                  num_t, den_t, att_v, sem0, sem1):
    cid = lax.axis_index("c")
    sid = lax.axis_index("s")
    wid = sid * NC + cid

    pltpu.sync_copy(att_hbm, att_v)
    att = [att_v[pl.ds(L * j, L)] for j in range(HD // L)]
    lane = lax.iota(jnp.int32, L)
    perms = [lane ^ (1 << p) for p in range(4)]
    gdn = lax.GatherDimensionNumbers(offset_dims=(), collapsed_slice_dims=(0,),
                                     start_index_map=(0,))

    def _allsum(v):
        for p in perms:
            v = v + lax.gather(v, p[:, None], gdn, (1,),
                               mode=lax.GatherScatterMode.PROMISE_IN_BOUNDS)
        return v

    zf = jnp.zeros((L,), jnp.float32)

    def zrow(r, _):
        for j in range(HD // L):
            num_t[r, pl.ds(L * j, L)] = zf
        den_t[r, :] = zf
        return 0

    lax.fori_loop(0, NPT + 1, zrow, 0)

    def fchunk(fc, cnt):
        pltpu.sync_copy(src_hbm.at[pl.ds(fc * FCH, FCH)], sbuf)
        pltpu.sync_copy(dst_hbm.at[pl.ds(fc * FCH, FCH)], dbuf)

        def fvec(i, cnt):
            s = sbuf[pl.ds(i * L, L)]
            dg = dbuf[pl.ds(i * L, L)]
            msk = (dg >> 6) == wid
            key = jnp.where(msk, lane, lane + L)
            _, vs = plsc.sort_key_val(key, s | (dg << 11))
            pkl[pl.ds(cnt, L)] = vs
            return cnt + plsc.all_reduce_population_count(msk)[0]

        return lax.fori_loop(0, FCH // L, fvec, cnt)

    cnt = lax.fori_loop(0, E // FCH, fchunk, jnp.int32(0))

    pad_v = jnp.full((L,), 4095 << 11, jnp.int32)
    for t in range(K // L):
        pkl[pl.ds(cnt + L * t, L)] = pad_v

    wbase = wid * NPT

    def chunk_body(ci, _):
        co = ci * K
        for j in range(K // L):
            v = pkl[pl.ds(co + L * j, L)]
            dg = v >> 11
            srcc[pl.ds(L * j, L)] = v & 2047
            dstc[pl.ds(L * j, L)] = jnp.minimum(dg, N - 1)
            dlc[pl.ds(L * j, L)] = jnp.minimum(dg - wbase, NPT)
        cl = pltpu.async_copy(xl_hbm.at[srcc], xl_rows, sem0)
        cr = pltpu.async_copy(xr_hbm.at[dstc], xr_rows, sem1)
        cl.wait()
        cr.wait()

        def edge_body(k, _):
            dl = dlc[pl.ds(k, L)][0]
            a = [xl_rows[k, pl.ds(L * j, L)] for j in range(HD // L)]
            b = [xr_rows[k, pl.ds(L * j, L)] for j in range(HD // L)]
            w = []
            for h in range(H):
                acc = jnp.zeros((L,), jnp.float32)
                for j in range(H * h, H * (h + 1)):
                    m = a[j] + b[j]
                    lm = jnp.maximum(m, SLOPE * m)
                    acc = acc + _rne_bf16(lm) * att[j]
                w.append(jnp.exp(_allsum(acc)))
            for h in range(H):
                for j in range(H * h, H * (h + 1)):
                    plsc.addupdate(num_t.at[dl, pl.ds(L * j, L)], a[j] * w[h])
            dv = jnp.where(lane == 0, w[0], 0.0)
            dv = jnp.where(lane == 1, w[1], dv)
            dv = jnp.where(lane == 2, w[2], dv)
            dv = jnp.where(lane == 3, w[3], dv)
            plsc.addupdate(den_t.at[dl, :], dv)
            return 0

        lax.fori_loop(0, K, edge_body, 0)
        return 0

    nch = (cnt + K - 1) // K
    lax.fori_loop(0, nch, chunk_body, 0)

    pltpu.sync_copy(num_t.at[pl.ds(0, NPT)], num_out.at[pl.ds(wid * NPT, NPT)])
    pltpu.sync_copy(den_t.at[pl.ds(0, NPT)], den_out.at[pl.ds(wid * NPT, NPT)])


def _edge_sc(src, dst, xl, xr, att_r):
    mesh = plsc.VectorSubcoreMesh(core_axis_name="c", subcore_axis_name="s")
    k = pl.kernel(
        _edge_sc_body,
        out_type=[
            jax.ShapeDtypeStruct((N, HD), jnp.float32),
            jax.ShapeDtypeStruct((N, L), jnp.float32),
        ],
        mesh=mesh,
        compiler_params=pltpu.CompilerParams(needs_layout_passes=False),
        scratch_types=[
            pltpu.VMEM((FCH,), jnp.int32),
            pltpu.VMEM((FCH,), jnp.int32),
            pltpu.VMEM((LMAX,), jnp.int32),
            pltpu.VMEM((K,), jnp.int32),
            pltpu.VMEM((K,), jnp.int32),
            pltpu.VMEM((K + L,), jnp.int32),
            pltpu.VMEM((K, HD), jnp.float32),
            pltpu.VMEM((K, HD), jnp.float32),
            pltpu.VMEM((NPT + 1, HD), jnp.float32),
            pltpu.VMEM((NPT + 1, L), jnp.float32),
            pltpu.VMEM((HD,), jnp.float32),
            pltpu.SemaphoreType.DMA,
            pltpu.SemaphoreType.DMA,
        ],
    )
    return k(src, dst, xl, xr, att_r)


def _combine_body(num_ref, den_ref, gb_ref, out_ref):
    num = num_ref[...]
    den = den_ref[...]
    acc = jnp.zeros((CB, D), jnp.float32)
    for h in range(H):
        nh = num[:, h * D:(h + 1) * D]
        dh = den[:, h:h + 1] + 1e-16
        acc = acc + nh / dh
    out = acc * (1.0 / H) + gb_ref[...]
    out_ref[...] = jnp.maximum(out, SLOPE * out)


def _combine(num, den, gat_bias):
    return pl.pallas_call(
        _combine_body,
        grid=(N // CB,),
        in_specs=[
            pl.BlockSpec((CB, HD), lambda i: (i, 0)),
            pl.BlockSpec((CB, L), lambda i: (i, 0)),
            pl.BlockSpec((1, D), lambda i: (0, 0)),
        ],
        out_specs=pl.BlockSpec((CB, D), lambda i: (i, 0)),
        out_shape=jax.ShapeDtypeStruct((N, D), jnp.float32),
    )(num, den, gat_bias.reshape(1, D))


def _mlp_body(flat_ref, w1_ref, b1_ref, w2_ref, b2_ref, w3_ref, b3_ref,
              y_ref, acc_ref):
    i = pl.program_id(0)
    fb = flat_ref[...].astype(jnp.bfloat16)
    part = jnp.dot(fb, w1_ref[...].astype(jnp.bfloat16),
                   preferred_element_type=jnp.float32)

    @pl.when(i == 0)
    def _():
        acc_ref[...] = jnp.zeros_like(acc_ref)

    acc_ref[...] += part

    @pl.when(i == pl.num_programs(0) - 1)
    def _():
        h = jnp.maximum(acc_ref[...] + b1_ref[...], 0.0).astype(jnp.bfloat16)
        h2 = jnp.dot(h, w2_ref[...].astype(jnp.bfloat16),
                     preferred_element_type=jnp.float32) + b2_ref[...]
        h2 = jnp.maximum(h2, 0.0).astype(jnp.bfloat16)
        y_ref[...] = jnp.dot(h2, w3_ref[...].astype(jnp.bfloat16),
                             preferred_element_type=jnp.float32) + b3_ref[...]


def _mlp(flat, w1_bf, b1, w2_bf, b2, w3_bf, b3):
    y = pl.pallas_call(
        _mlp_body,
        grid=(N * D // FB,),
        in_specs=[
            pl.BlockSpec((1, FB), lambda i: (0, i)),
            pl.BlockSpec((FB, H1), lambda i: (i, 0)),
            pl.BlockSpec((1, H1), lambda i: (0, 0)),
            pl.BlockSpec((H1, H2), lambda i: (0, 0)),
            pl.BlockSpec((1, H2), lambda i: (0, 0)),
            pl.BlockSpec((H2, 1), lambda i: (0, 0)),
            pl.BlockSpec((1, 1), lambda i: (0, 0)),
        ],
        out_specs=pl.BlockSpec((1, 1), lambda i: (0, 0)),
        out_shape=jax.ShapeDtypeStruct((1, 1), jnp.float32),
        scratch_shapes=[pltpu.VMEM((1, H1), jnp.float32)],
    )(flat.reshape(1, N * D), w1_bf, b1.reshape(1, H1),
      w2_bf, b2.reshape(1, H2), w3_bf, b3.reshape(1, 1))
    return y.reshape(1)


def kernel(protein_embeddings, edge_index, W_in, b_in, Wl, bl, Wr, br, att,
           gat_bias, W1, b1, W2, b2, W3, b3):
    src = edge_index[0].astype(jnp.int32)
    dst = edge_index[1].astype(jnp.int32)
    bf = jnp.bfloat16
    xl, xr = _embed(protein_embeddings.astype(bf), W_in, b_in,
                    Wl.astype(bf), bl, Wr.astype(bf), br)
    att_r = att.astype(bf).astype(jnp.float32).reshape(HD)
    num, den = _edge_sc(src, dst, xl, xr, att_r)
    out = _combine(num, den, gat_bias)
    return _mlp(out, W1, b1, W2, b2, W3, b3)
